# baseline (speedup 1.0000x reference)
import functools

import jax
import jax.numpy as jnp
from jax.experimental import pallas as pl
from jax.experimental.pallas import tpu as pltpu

B = 1024
N = 100
D = 128
G = 256
GI = 32
NCH = G // GI
CROWS = GI * N
ROWS = G * N
GRID = B // G


def _body(f_ref, sel_ref, phq_ref, Wphk_ref, bphk_ref, Wphv_ref, bphv_ref,
          anq_ref, Wank_ref, bank_ref, Wanv_ref, banv_ref,
          h_ref, wp_ref, wa_ref):
    phq = phq_ref[...]
    anq = anq_ref[...]
    dn = (((1,), (0,)), ((), ()))
    dnt = (((1,), (1,)), ((), ()))

    qph = jax.lax.dot_general(phq, Wphk_ref[...], dn,
                              preferred_element_type=jnp.float32)
    cph = jnp.sum(phq * bphk_ref[...], axis=1, keepdims=True)
    AQ = jax.lax.dot_general(anq, Wank_ref[...], dn,
                             preferred_element_type=jnp.float32)
    can4 = jnp.sum(anq * bank_ref[...], axis=1, keepdims=True)

    q8 = jnp.concatenate(
        [AQ, qph, jnp.zeros((3, D), jnp.float32)], axis=0)
    c8 = jnp.concatenate(
        [can4, cph, jnp.zeros((3, 1), jnp.float32)], axis=0)

    rlane = jax.lax.broadcasted_iota(jnp.int32, (2 * GI, CROWS), 1)
    gsub = jax.lax.broadcasted_iota(jnp.int32, (2 * GI, CROWS), 0)
    seg2 = (rlane // N == gsub % GI).astype(jnp.float32)
    ones = jnp.ones((CROWS, 1), jnp.float32)

    s_l, ws_l = [], []
    for c in range(NCH):
        fc = f_ref[c * CROWS:(c + 1) * CROWS, :]
        L = jax.lax.dot_general(fc, q8, dnt,
                                preferred_element_type=jnp.float32)
        Lt = L.T
        wfull = jax.nn.sigmoid(Lt + c8)
        selc = sel_ref[:, c * CROWS:(c + 1) * CROWS]
        wa_t = jnp.sum(wfull[0:4] * selc, axis=0, keepdims=True)
        wp_t = wfull[4:5]
        wp_ref[0, c, :] = wp_t.reshape(CROWS)
        wa_ref[0, c, :] = wa_t.reshape(CROWS)
        W2 = seg2 * jnp.concatenate(
            [jnp.broadcast_to(wp_t, (GI, CROWS)),
             jnp.broadcast_to(wa_t, (GI, CROWS))], axis=0)
        s_l.append(jax.lax.dot_general(W2, fc, dn,
                                       preferred_element_type=jnp.float32))
        ws_l.append(jax.lax.dot_general(W2, ones, dn,
                                        preferred_element_type=jnp.float32))

    sph = jnp.concatenate([s[0:GI] for s in s_l], axis=0)
    san = jnp.concatenate([s[GI:2 * GI] for s in s_l], axis=0)
    wsp = jnp.concatenate([w[0:GI] for w in ws_l], axis=0)
    wsa = jnp.concatenate([w[GI:2 * GI] for w in ws_l], axis=0)
    h_ref[...] = (jax.lax.dot_general(sph, Wphv_ref[...], dnt,
                                      preferred_element_type=jnp.float32)
                  + wsp * bphv_ref[...]
                  + jax.lax.dot_general(san, Wanv_ref[...], dnt,
                                        preferred_element_type=jnp.float32)
                  + wsa * banv_ref[...])


@functools.partial(jax.jit, static_argnames=())
def kernel(feats, ancestries, W_phk, b_phk, W_phv, b_phv, ph_query,
           W_ank, b_ank, W_anv, b_anv, ancestry_query):
    oh = (jnp.arange(4, dtype=jnp.int32)[:, None] == ancestries[None, :]
          ).astype(jnp.float32)
    sel = jnp.broadcast_to(oh[:, :, None], (4, B, N)).reshape(4, B * N)
    full = lambda shape: pl.BlockSpec(shape, lambda i: (0, 0))
    h, wp, wa = pl.pallas_call(
        _body,
        grid=(GRID,),
        in_specs=[
            pl.BlockSpec((ROWS, D), lambda i: (i, 0)),
            pl.BlockSpec((4, ROWS), lambda i: (0, i)),
            full((1, D)),
            full((D, D)),
            full((1, D)),
            full((D, D)),
            full((1, D)),
            full((4, D)),
            full((D, D)),
            full((1, D)),
            full((D, D)),
            full((1, D)),
        ],
        out_specs=[
            pl.BlockSpec((G, D), lambda i: (i, 0)),
            pl.BlockSpec((1, NCH, CROWS), lambda i: (i, 0, 0)),
            pl.BlockSpec((1, NCH, CROWS), lambda i: (i, 0, 0)),
        ],
        out_shape=[
            jax.ShapeDtypeStruct((B, D), jnp.float32),
            jax.ShapeDtypeStruct((GRID, NCH, CROWS), jnp.float32),
            jax.ShapeDtypeStruct((GRID, NCH, CROWS), jnp.float32),
        ],
        compiler_params=pltpu.CompilerParams(
            dimension_semantics=("parallel",)),
    )(feats, sel, ph_query, W_phk, b_phk.reshape(1, D), W_phv,
      b_phv.reshape(1, D), ancestry_query, W_ank, b_ank.reshape(1, D),
      W_anv, b_anv.reshape(1, D))
    return (h, wp.reshape(B * N, 1), wa.reshape(B * N, 1))


# speedup vs baseline: 1.1946x; 1.1946x over previous
import functools

import jax
import jax.numpy as jnp
from jax.experimental import pallas as pl
from jax.experimental.pallas import tpu as pltpu

B = 1024
N = 100
D = 128
G = 256
GI = 64
NCH = G // GI
CROWS = GI * N
ROWS = G * N
GRID = B // G


def _body(f_ref, oh_ref, phq_ref, Wphk_ref, bphk_ref, Wphv_ref, bphv_ref,
          anq_ref, Wank_ref, bank_ref, Wanv_ref, banv_ref,
          h_ref, wp_ref, wa_ref):
    f = f_ref[...]
    phq = phq_ref[...]
    anq = anq_ref[...]
    dn = (((1,), (0,)), ((), ()))
    dnt = (((1,), (1,)), ((), ()))

    qph = jax.lax.dot_general(phq, Wphk_ref[...], dn,
                              preferred_element_type=jnp.float32)
    cph = jnp.sum(phq * bphk_ref[...])
    AQ = jax.lax.dot_general(anq, Wank_ref[...], dn,
                             preferred_element_type=jnp.float32)
    can4 = jnp.sum(anq * bank_ref[...], axis=1, keepdims=True)
    oh = oh_ref[...]
    qa = jax.lax.dot_general(oh, AQ, dn,
                             preferred_element_type=jnp.float32)
    can = jax.lax.dot_general(oh, can4, dn,
                              preferred_element_type=jnp.float32)

    rlane = jax.lax.broadcasted_iota(jnp.int32, (GI, CROWS), 1)
    gsub = jax.lax.broadcasted_iota(jnp.int32, (GI, CROWS), 0)
    seg = (rlane // N == gsub).astype(jnp.float32)
    pad = jnp.zeros(((-(GI + 1)) % 8, D), jnp.float32)

    sph_l, san_l, wsp_l, wsa_l = [], [], [], []
    for c in range(NCH):
        fc = f[c * CROWS:(c + 1) * CROWS]
        qac = qa[c * GI:(c + 1) * GI]
        canc = can[c * GI:(c + 1) * GI]
        q16 = jnp.concatenate([qac, qph, pad], axis=0)
        L = jax.lax.dot_general(fc, q16, dnt,
                                preferred_element_type=jnp.float32)
        Lt = L.T
        la_t = jnp.sum((Lt[0:GI] + canc) * seg, axis=0,
                       keepdims=True)
        lp_t = Lt[GI:GI + 1] + cph
        wp_t = jax.nn.sigmoid(lp_t)
        wa_t = jax.nn.sigmoid(la_t)
        wp_ref[0, c, :] = wp_t.reshape(CROWS)
        wa_ref[0, c, :] = wa_t.reshape(CROWS)
        Wp = seg * wp_t
        Wa = seg * wa_t
        sph_l.append(jax.lax.dot_general(Wp, fc, dn,
                                         preferred_element_type=jnp.float32))
        san_l.append(jax.lax.dot_general(Wa, fc, dn,
                                         preferred_element_type=jnp.float32))
        wsp_l.append(jnp.sum(Wp, axis=1, keepdims=True))
        wsa_l.append(jnp.sum(Wa, axis=1, keepdims=True))

    sph = jnp.concatenate(sph_l, axis=0)
    san = jnp.concatenate(san_l, axis=0)
    wsp = jnp.concatenate(wsp_l, axis=0)
    wsa = jnp.concatenate(wsa_l, axis=0)
    h_ref[...] = (jax.lax.dot_general(sph, Wphv_ref[...], dnt,
                                      preferred_element_type=jnp.float32)
                  + wsp * bphv_ref[...]
                  + jax.lax.dot_general(san, Wanv_ref[...], dnt,
                                        preferred_element_type=jnp.float32)
                  + wsa * banv_ref[...])


@functools.partial(jax.jit, static_argnames=())
def kernel(feats, ancestries, W_phk, b_phk, W_phv, b_phv, ph_query,
           W_ank, b_ank, W_anv, b_anv, ancestry_query):
    oh = (ancestries[:, None] == jnp.arange(4, dtype=jnp.int32)[None, :]
          ).astype(jnp.float32)
    full = lambda shape: pl.BlockSpec(shape, lambda i: (0, 0))
    h, wp, wa = pl.pallas_call(
        _body,
        grid=(GRID,),
        in_specs=[
            pl.BlockSpec((ROWS, D), lambda i: (i, 0)),
            pl.BlockSpec((G, 4), lambda i: (i, 0)),
            full((1, D)),
            full((D, D)),
            full((1, D)),
            full((D, D)),
            full((1, D)),
            full((4, D)),
            full((D, D)),
            full((1, D)),
            full((D, D)),
            full((1, D)),
        ],
        out_specs=[
            pl.BlockSpec((G, D), lambda i: (i, 0)),
            pl.BlockSpec((1, NCH, CROWS), lambda i: (i, 0, 0)),
            pl.BlockSpec((1, NCH, CROWS), lambda i: (i, 0, 0)),
        ],
        out_shape=[
            jax.ShapeDtypeStruct((B, D), jnp.float32),
            jax.ShapeDtypeStruct((GRID, NCH, CROWS), jnp.float32),
            jax.ShapeDtypeStruct((GRID, NCH, CROWS), jnp.float32),
        ],
        compiler_params=pltpu.CompilerParams(
            dimension_semantics=("parallel",)),
    )(feats, oh, ph_query, W_phk, b_phk.reshape(1, D), W_phv,
      b_phv.reshape(1, D), ancestry_query, W_ank, b_ank.reshape(1, D),
      W_anv, b_anv.reshape(1, D))
    return (h, wp.reshape(B * N, 1), wa.reshape(B * N, 1))
